# Initial kernel scaffold; baseline (speedup 1.0000x reference)
import jax
import jax.numpy as jnp
from jax.experimental import pallas as pl


def kernel(scores):
    raise NotImplementedError("write your pallas kernel here")


# speedup vs baseline: 6.5133x; 6.5133x over previous
import jax
import jax.numpy as jnp
from jax.experimental import pallas as pl
from jax.experimental.pallas import tpu as pltpu

_K = 32
_R = 256


def _body(x_ref, adj_ref, ew_ref, ei_ref, ll_ref):
    i = pl.program_id(0)
    x = x_ref[...]
    r, n = x.shape

    m = jnp.max(x, axis=-1, keepdims=True)
    s = jnp.sum(jnp.exp(x - m), axis=-1, keepdims=True)
    lse = m + jnp.log(s)

    b = jax.lax.bitcast_convert_type(x, jnp.int32)
    v = b ^ ((b >> 31) & jnp.int32(0x7FFFFFFF))

    lo0 = jnp.full((r, 1), jnp.iinfo(jnp.int32).min, dtype=jnp.int32)
    hi0 = jnp.full((r, 1), jnp.iinfo(jnp.int32).max, dtype=jnp.int32)

    def step(_, carry):
        lo, hi = carry
        mid = lo + jax.lax.shift_right_logical(hi - lo, 1)
        cnt = jnp.sum((v > mid).astype(jnp.int32), axis=-1, keepdims=True)
        big = cnt >= _K
        return jnp.where(big, mid + 1, lo), jnp.where(big, hi, mid)

    lo, _hi = jax.lax.fori_loop(0, 32, step, (lo0, hi0))

    mask = (v >= lo).astype(jnp.float32)
    adj_ref[...] = mask
    ew_ref[...] = mask

    li = jax.lax.broadcasted_iota(jnp.int32, (r, n), 0) + i * r
    cj = jax.lax.broadcasted_iota(jnp.int32, (r, n), 1)
    ei_ref[0] = li
    ei_ref[1] = (li // n) * n + cj

    cnt = jnp.sum(mask, axis=-1)
    msum = jnp.sum(mask * x, axis=-1)
    ll_ref[...] = msum - cnt * lse[:, 0]


def kernel(scores):
    bsz, n, n2 = scores.shape
    rtot = bsz * n
    r = _R if rtot % _R == 0 else rtot
    grid = rtot // r
    x2 = scores.reshape(rtot, n2)
    adj2, ew2, ei3, ll1 = pl.pallas_call(
        _body,
        grid=(grid,),
        in_specs=[pl.BlockSpec((r, n2), lambda i: (i, 0))],
        out_specs=[
            pl.BlockSpec((r, n2), lambda i: (i, 0)),
            pl.BlockSpec((r, n2), lambda i: (i, 0)),
            pl.BlockSpec((2, r, n2), lambda i: (0, i, 0)),
            pl.BlockSpec((r,), lambda i: (i,)),
        ],
        out_shape=[
            jax.ShapeDtypeStruct((rtot, n2), jnp.float32),
            jax.ShapeDtypeStruct((rtot, n2), jnp.float32),
            jax.ShapeDtypeStruct((2, rtot, n2), jnp.int32),
            jax.ShapeDtypeStruct((rtot,), jnp.float32),
        ],
        compiler_params=pltpu.CompilerParams(
            dimension_semantics=("arbitrary",)),
    )(x2)
    return (
        adj2.reshape(bsz, n, n2),
        ei3.reshape(2, rtot * n2),
        ew2.reshape(rtot * n2),
        ll1.reshape(bsz, n),
    )
